# Initial kernel scaffold; baseline (speedup 1.0000x reference)
import jax
import jax.numpy as jnp
from jax.experimental import pallas as pl


def kernel(inputs, grid, bounds):
    raise NotImplementedError("write your pallas kernel here")


# speedup vs baseline: 1.0113x; 1.0113x over previous
import functools

import jax
import jax.numpy as jnp
from jax import lax
from jax.experimental import pallas as pl
from jax.experimental.pallas import tpu as pltpu
from jax.experimental.pallas import tpu_sc as plsc

NC = 2
NS = 16
NW = NC * NS
L = 16


def _make_sc_interp(n, h, w):
    per_w = n // NW
    t = 2048
    nb = per_w // t
    mesh = plsc.VectorSubcoreMesh(core_axis_name="c", subcore_axis_name="s")

    @functools.partial(
        pl.kernel,
        mesh=mesh,
        out_type=jax.ShapeDtypeStruct((n,), jnp.float32),
        scratch_types=[
            pltpu.VMEM((4, L), jnp.float32),
            pltpu.VMEM((t,), jnp.float32),
            pltpu.VMEM((t,), jnp.float32),
            pltpu.VMEM((4 * t,), jnp.int32),
            pltpu.VMEM((4 * t,), jnp.float32),
            pltpu.VMEM((t,), jnp.float32),
            pltpu.VMEM((t,), jnp.float32),
            pltpu.VMEM((t,), jnp.float32),
            pltpu.SemaphoreType.DMA,
        ],
    )
    def kern(x1_hbm, x2_hbm, table_hbm, params_hbm, out_hbm,
             params_v, x1_v, x2_v, idx_v, vals_v, ay_v, ax_v, out_v, sem):
        cid = lax.axis_index("c")
        sid = lax.axis_index("s")
        wid = sid * NC + cid
        base_w = wid * per_w

        pltpu.sync_copy(params_hbm, params_v)
        sy = params_v[0]
        sx = params_v[1]
        oy = params_v[2]
        ox = params_v[3]

        def block(b, carry):
            off = base_w + b * t
            pltpu.sync_copy(x1_hbm.at[pl.ds(off, t)], x1_v)
            pltpu.sync_copy(x2_hbm.at[pl.ds(off, t)], x2_v)

            def compute(j, carry2):
                s = j * L
                x1 = x1_v[pl.ds(s, L)]
                x2 = x2_v[pl.ds(s, L)]
                qy = jnp.maximum(x1 * sy + oy, 0.0)
                qx = jnp.maximum(x2 * sx + ox, 0.0)
                fy = jnp.minimum(qy.astype(jnp.int32), h - 2)
                fx = jnp.minimum(qx.astype(jnp.int32), w - 2)
                ay = jnp.minimum(qy - fy.astype(jnp.float32), 1.0)
                ax = jnp.minimum(qx - fx.astype(jnp.float32), 1.0)
                lin = fy * w + fx
                idx_v[pl.ds(s, L)] = lin
                idx_v[pl.ds(t + s, L)] = lin + 1
                idx_v[pl.ds(2 * t + s, L)] = lin + w
                idx_v[pl.ds(3 * t + s, L)] = lin + (w + 1)
                ay_v[pl.ds(s, L)] = ay
                ax_v[pl.ds(s, L)] = ax
                return carry2

            lax.fori_loop(0, t // L, compute, 0, unroll=4)

            pltpu.async_copy(table_hbm.at[idx_v], vals_v, sem).wait()

            def combine(j, carry2):
                s = j * L
                tl = vals_v[pl.ds(s, L)]
                tr = vals_v[pl.ds(t + s, L)]
                bl = vals_v[pl.ds(2 * t + s, L)]
                br = vals_v[pl.ds(3 * t + s, L)]
                ax = ax_v[pl.ds(s, L)]
                ay = ay_v[pl.ds(s, L)]
                top = ax * (tr - tl) + tl
                bot = ax * (br - bl) + bl
                out_v[pl.ds(s, L)] = ay * (bot - top) + top
                return carry2

            lax.fori_loop(0, t // L, combine, 0, unroll=4)

            pltpu.sync_copy(out_v, out_hbm.at[pl.ds(off, t)])
            return carry

        lax.fori_loop(0, nb, block, 0)

    return kern


def kernel(inputs, grid, bounds):
    n = inputs.shape[0]
    _, h, w, _ = grid.shape
    scale = (jnp.array([h, w], jnp.float32) - 1.0) / (bounds[1] - bounds[0])
    off = -bounds[0] * scale
    params = jnp.concatenate([scale, off]).reshape(4, 1)
    params = jnp.broadcast_to(params, (4, L)).astype(jnp.float32)
    planes = inputs.T
    table = grid.reshape(-1)
    out = _make_sc_interp(n, h, w)(planes[0], planes[1], table, params)
    return out.reshape(n, 1)


# speedup vs baseline: 1.2477x; 1.2338x over previous
import functools

import jax
import jax.numpy as jnp
from jax import lax
from jax.experimental import pallas as pl
from jax.experimental.pallas import tpu as pltpu
from jax.experimental.pallas import tpu_sc as plsc

NC = 2
NS = 16
NW = NC * NS
L = 16


def _make_sc_interp(n, h, w):
    per_w = n // NW
    t = 4096
    nb = per_w // t
    mesh = plsc.VectorSubcoreMesh(core_axis_name="c", subcore_axis_name="s")

    @functools.partial(
        pl.kernel,
        mesh=mesh,
        out_type=jax.ShapeDtypeStruct((n,), jnp.float32),
        scratch_types=[
            pltpu.VMEM((4, L), jnp.float32),
            pltpu.VMEM((t,), jnp.float32), pltpu.VMEM((t,), jnp.float32),
            pltpu.VMEM((t,), jnp.float32), pltpu.VMEM((t,), jnp.float32),
            pltpu.VMEM((4 * t,), jnp.int32), pltpu.VMEM((4 * t,), jnp.int32),
            pltpu.VMEM((4 * t,), jnp.float32), pltpu.VMEM((4 * t,), jnp.float32),
            pltpu.VMEM((t,), jnp.float32), pltpu.VMEM((t,), jnp.float32),
            pltpu.VMEM((t,), jnp.float32), pltpu.VMEM((t,), jnp.float32),
            pltpu.VMEM((t,), jnp.float32),
            pltpu.SemaphoreType.DMA,
            pltpu.SemaphoreType.DMA,
        ],
    )
    def kern(x1_hbm, x2_hbm, table_hbm, params_hbm, out_hbm,
             params_v, x1_a, x1_b, x2_a, x2_b, idx_a, idx_b,
             vals_a, vals_b, ay_a, ay_b, ax_a, ax_b, out_v,
             sem0, sem1):
        cid = lax.axis_index("c")
        sid = lax.axis_index("s")
        wid = sid * NC + cid
        base_w = wid * per_w
        x1s, x2s = (x1_a, x1_b), (x2_a, x2_b)
        idxs, valss = (idx_a, idx_b), (vals_a, vals_b)
        ays, axs = (ay_a, ay_b), (ax_a, ax_b)
        sems = (sem0, sem1)

        pltpu.sync_copy(params_hbm, params_v)
        sy = params_v[0]
        sx = params_v[1]
        oy = params_v[2]
        ox = params_v[3]

        def load_inputs(b, p):
            off = base_w + b * t
            pltpu.sync_copy(x1_hbm.at[pl.ds(off, t)], x1s[p])
            pltpu.sync_copy(x2_hbm.at[pl.ds(off, t)], x2s[p])

        def compute_idx(p):
            x1_v, x2_v, idx_v = x1s[p], x2s[p], idxs[p]
            ay_v, ax_v = ays[p], axs[p]

            def body(j, carry):
                s = j * L
                x1 = x1_v[pl.ds(s, L)]
                x2 = x2_v[pl.ds(s, L)]
                qy = jnp.maximum(x1 * sy + oy, 0.0)
                qx = jnp.maximum(x2 * sx + ox, 0.0)
                fy = jnp.minimum(qy.astype(jnp.int32), h - 2)
                fx = jnp.minimum(qx.astype(jnp.int32), w - 2)
                ay = jnp.minimum(qy - fy.astype(jnp.float32), 1.0)
                ax = jnp.minimum(qx - fx.astype(jnp.float32), 1.0)
                lin = fy * w + fx
                idx_v[pl.ds(s, L)] = lin
                idx_v[pl.ds(t + s, L)] = lin + 1
                idx_v[pl.ds(2 * t + s, L)] = lin + w
                idx_v[pl.ds(3 * t + s, L)] = lin + (w + 1)
                ay_v[pl.ds(s, L)] = ay
                ax_v[pl.ds(s, L)] = ax
                return carry

            lax.fori_loop(0, t // L, body, 0, unroll=4)

        def start_gather(p):
            return pltpu.async_copy(table_hbm.at[idxs[p]], valss[p], sems[p])

        def combine_store(b, p):
            vals_v, ay_v, ax_v = valss[p], ays[p], axs[p]

            def body(j, carry):
                s = j * L
                tl = vals_v[pl.ds(s, L)]
                tr = vals_v[pl.ds(t + s, L)]
                bl = vals_v[pl.ds(2 * t + s, L)]
                br = vals_v[pl.ds(3 * t + s, L)]
                ax = ax_v[pl.ds(s, L)]
                ay = ay_v[pl.ds(s, L)]
                top = ax * (tr - tl) + tl
                bot = ax * (br - bl) + bl
                out_v[pl.ds(s, L)] = ay * (bot - top) + top
                return carry

            lax.fori_loop(0, t // L, body, 0, unroll=4)
            pltpu.sync_copy(out_v, out_hbm.at[pl.ds(base_w + b * t, t)])

        load_inputs(0, 0)
        compute_idx(0)
        handles = {0: start_gather(0)}
        for b in range(1, nb):
            p = b % 2
            load_inputs(b, p)
            compute_idx(p)
            handles[b] = start_gather(p)
            handles[b - 1].wait()
            combine_store(b - 1, (b - 1) % 2)
        handles[nb - 1].wait()
        combine_store(nb - 1, (nb - 1) % 2)

    return kern


def kernel(inputs, grid, bounds):
    n = inputs.shape[0]
    _, h, w, _ = grid.shape
    scale = (jnp.array([h, w], jnp.float32) - 1.0) / (bounds[1] - bounds[0])
    off = -bounds[0] * scale
    params = jnp.concatenate([scale, off]).reshape(4, 1)
    params = jnp.broadcast_to(params, (4, L)).astype(jnp.float32)
    planes = inputs.T
    table = grid.reshape(-1)
    out = _make_sc_interp(n, h, w)(planes[0], planes[1], table, params)
    return out.reshape(n, 1)


# speedup vs baseline: 1.2486x; 1.0008x over previous
import functools

import jax
import jax.numpy as jnp
from jax import lax
from jax.experimental import pallas as pl
from jax.experimental.pallas import tpu as pltpu
from jax.experimental.pallas import tpu_sc as plsc

NC = 2
NS = 16
NW = NC * NS
L = 16


def _make_sc_interp(n, h, w):
    per_w = n // NW
    t = 4096
    nb = per_w // t
    mesh = plsc.VectorSubcoreMesh(core_axis_name="c", subcore_axis_name="s")

    @functools.partial(
        pl.kernel,
        mesh=mesh,
        out_type=jax.ShapeDtypeStruct((n,), jnp.float32),
        scratch_types=[
            pltpu.VMEM((4, L), jnp.float32),
            pltpu.VMEM((t,), jnp.float32), pltpu.VMEM((t,), jnp.float32),
            pltpu.VMEM((t,), jnp.float32), pltpu.VMEM((t,), jnp.float32),
            pltpu.VMEM((4 * t,), jnp.int32), pltpu.VMEM((4 * t,), jnp.int32),
            pltpu.VMEM((4 * t,), jnp.float32), pltpu.VMEM((4 * t,), jnp.float32),
            pltpu.VMEM((t,), jnp.float32), pltpu.VMEM((t,), jnp.float32),
            pltpu.VMEM((t,), jnp.float32), pltpu.VMEM((t,), jnp.float32),
            pltpu.VMEM((t,), jnp.float32),
            pltpu.SemaphoreType.DMA,
            pltpu.SemaphoreType.DMA,
        ],
    )
    def kern(x1_hbm, x2_hbm, table_hbm, params_hbm, out_hbm,
             params_v, x1_a, x1_b, x2_a, x2_b, idx_a, idx_b,
             vals_a, vals_b, ay_a, ay_b, ax_a, ax_b, out_v,
             sem0, sem1):
        cid = lax.axis_index("c")
        sid = lax.axis_index("s")
        wid = sid * NC + cid
        base_w = wid * per_w
        x1s, x2s = (x1_a, x1_b), (x2_a, x2_b)
        idxs, valss = (idx_a, idx_b), (vals_a, vals_b)
        ays, axs = (ay_a, ay_b), (ax_a, ax_b)
        sems = (sem0, sem1)

        pltpu.sync_copy(params_hbm, params_v)
        sy = params_v[0]
        sx = params_v[1]
        oy = params_v[2]
        ox = params_v[3]

        def load_inputs(b, p):
            off = base_w + b * t
            pltpu.sync_copy(x1_hbm.at[pl.ds(off, t)], x1s[p])
            pltpu.sync_copy(x2_hbm.at[pl.ds(off, t)], x2s[p])

        def compute_idx(p):
            x1_v, x2_v, idx_v = x1s[p], x2s[p], idxs[p]
            ay_v, ax_v = ays[p], axs[p]

            def body(j, carry):
                s = j * L
                x1 = x1_v[pl.ds(s, L)]
                x2 = x2_v[pl.ds(s, L)]
                qy = jnp.maximum(x1 * sy + oy, 0.0)
                qx = jnp.maximum(x2 * sx + ox, 0.0)
                fy = jnp.minimum(qy.astype(jnp.int32), h - 2)
                fx = jnp.minimum(qx.astype(jnp.int32), w - 2)
                ay = jnp.minimum(qy - fy.astype(jnp.float32), 1.0)
                ax = jnp.minimum(qx - fx.astype(jnp.float32), 1.0)
                lin = fy * w + fx
                idx_v[pl.ds(s, L)] = lin
                idx_v[pl.ds(t + s, L)] = lin + 1
                idx_v[pl.ds(2 * t + s, L)] = lin + w
                idx_v[pl.ds(3 * t + s, L)] = lin + (w + 1)
                ay_v[pl.ds(s, L)] = ay
                ax_v[pl.ds(s, L)] = ax
                return carry

            lax.fori_loop(0, t // L, body, 0, unroll=4)

        def start_gather(p):
            return pltpu.async_copy(table_hbm.at[idxs[p]], valss[p], sems[p])

        def combine_store(b, p):
            vals_v, ay_v, ax_v = valss[p], ays[p], axs[p]

            def body(j, carry):
                s = j * L
                tl = vals_v[pl.ds(s, L)]
                tr = vals_v[pl.ds(t + s, L)]
                bl = vals_v[pl.ds(2 * t + s, L)]
                br = vals_v[pl.ds(3 * t + s, L)]
                ax = ax_v[pl.ds(s, L)]
                ay = ay_v[pl.ds(s, L)]
                top = ax * (tr - tl) + tl
                bot = ax * (br - bl) + bl
                out_v[pl.ds(s, L)] = ay * (bot - top) + top
                return carry

            lax.fori_loop(0, t // L, body, 0, unroll=4)
            pltpu.sync_copy(out_v, out_hbm.at[pl.ds(base_w + b * t, t)])

        with jax.named_scope("prologue"):
            load_inputs(0, 0)
            compute_idx(0)
            handles = {0: start_gather(0)}
        for b in range(1, nb):
            p = b % 2
            with jax.named_scope("load_inputs"):
                load_inputs(b, p)
            with jax.named_scope("compute_idx"):
                compute_idx(p)
            handles[b] = start_gather(p)
            with jax.named_scope("gather_wait"):
                handles[b - 1].wait()
            with jax.named_scope("combine"):
                combine_store(b - 1, (b - 1) % 2)
        with jax.named_scope("gather_wait"):
            handles[nb - 1].wait()
        with jax.named_scope("combine"):
            combine_store(nb - 1, (nb - 1) % 2)

    return kern


def kernel(inputs, grid, bounds):
    n = inputs.shape[0]
    _, h, w, _ = grid.shape
    scale = (jnp.array([h, w], jnp.float32) - 1.0) / (bounds[1] - bounds[0])
    off = -bounds[0] * scale
    params = jnp.concatenate([scale, off]).reshape(4, 1)
    params = jnp.broadcast_to(params, (4, L)).astype(jnp.float32)
    planes = inputs.T
    table = grid.reshape(-1)
    out = _make_sc_interp(n, h, w)(planes[0], planes[1], table, params)
    return out.reshape(n, 1)


# speedup vs baseline: 1.3030x; 1.0436x over previous
import functools

import jax
import jax.numpy as jnp
from jax import lax
from jax.experimental import pallas as pl
from jax.experimental.pallas import tpu as pltpu
from jax.experimental.pallas import tpu_sc as plsc

NC = 2
NS = 16
NW = NC * NS
L = 16
ND = 4


def _make_sc_interp(n, h, w):
    per_w = n // NW
    t = 2048
    nb = per_w // t
    mesh = plsc.VectorSubcoreMesh(core_axis_name="c", subcore_axis_name="s")

    ring = lambda shp, dt: [pltpu.VMEM(shp, dt) for _ in range(ND)]

    @functools.partial(
        pl.kernel,
        mesh=mesh,
        out_type=jax.ShapeDtypeStruct((n,), jnp.float32),
        scratch_types=(
            [pltpu.VMEM((4, L), jnp.float32)]
            + ring((t,), jnp.float32) + ring((t,), jnp.float32)
            + ring((4 * t,), jnp.int32) + ring((4 * t,), jnp.float32)
            + ring((t,), jnp.float32) + ring((t,), jnp.float32)
            + [pltpu.VMEM((t,), jnp.float32)]
            + [pltpu.SemaphoreType.DMA] * ND
        ),
    )
    def kern(x1_hbm, x2_hbm, table_hbm, params_hbm, out_hbm, params_v, *sc):
        x1s, x2s = sc[0:ND], sc[ND:2 * ND]
        idxs, valss = sc[2 * ND:3 * ND], sc[3 * ND:4 * ND]
        ays, axs = sc[4 * ND:5 * ND], sc[5 * ND:6 * ND]
        out_v = sc[6 * ND]
        sems = sc[6 * ND + 1:6 * ND + 1 + ND]

        cid = lax.axis_index("c")
        sid = lax.axis_index("s")
        wid = sid * NC + cid
        base_w = wid * per_w

        pltpu.sync_copy(params_hbm, params_v)
        sy = params_v[0]
        sx = params_v[1]
        oy = params_v[2]
        ox = params_v[3]

        def load_inputs(b, p):
            off = base_w + b * t
            pltpu.sync_copy(x1_hbm.at[pl.ds(off, t)], x1s[p])
            pltpu.sync_copy(x2_hbm.at[pl.ds(off, t)], x2s[p])

        def compute_idx(p):
            x1_v, x2_v, idx_v = x1s[p], x2s[p], idxs[p]
            ay_v, ax_v = ays[p], axs[p]

            def body(j, carry):
                s = j * L
                x1 = x1_v[pl.ds(s, L)]
                x2 = x2_v[pl.ds(s, L)]
                qy = jnp.maximum(x1 * sy + oy, 0.0)
                qx = jnp.maximum(x2 * sx + ox, 0.0)
                fy = jnp.minimum(qy.astype(jnp.int32), h - 2)
                fx = jnp.minimum(qx.astype(jnp.int32), w - 2)
                ay = jnp.minimum(qy - fy.astype(jnp.float32), 1.0)
                ax = jnp.minimum(qx - fx.astype(jnp.float32), 1.0)
                lin = fy * w + fx
                idx_v[pl.ds(s, L)] = lin
                idx_v[pl.ds(t + s, L)] = lin + 1
                idx_v[pl.ds(2 * t + s, L)] = lin + w
                idx_v[pl.ds(3 * t + s, L)] = lin + (w + 1)
                ay_v[pl.ds(s, L)] = ay
                ax_v[pl.ds(s, L)] = ax
                return carry

            lax.fori_loop(0, t // L, body, 0, unroll=8)

        def start_gather(p):
            return pltpu.async_copy(table_hbm.at[idxs[p]], valss[p], sems[p])

        def combine_store(b, p):
            vals_v, ay_v, ax_v = valss[p], ays[p], axs[p]

            def body(j, carry):
                s = j * L
                tl = vals_v[pl.ds(s, L)]
                tr = vals_v[pl.ds(t + s, L)]
                bl = vals_v[pl.ds(2 * t + s, L)]
                br = vals_v[pl.ds(3 * t + s, L)]
                ax = ax_v[pl.ds(s, L)]
                ay = ay_v[pl.ds(s, L)]
                top = ax * (tr - tl) + tl
                bot = ax * (br - bl) + bl
                out_v[pl.ds(s, L)] = ay * (bot - top) + top
                return carry

            lax.fori_loop(0, t // L, body, 0, unroll=8)
            pltpu.sync_copy(out_v, out_hbm.at[pl.ds(base_w + b * t, t)])

        handles = {}
        for b in range(nb):
            p = b % ND
            if b >= ND:
                with jax.named_scope("gather_wait"):
                    handles[b - ND].wait()
                with jax.named_scope("combine"):
                    combine_store(b - ND, p)
            with jax.named_scope("load_inputs"):
                load_inputs(b, p)
            with jax.named_scope("compute_idx"):
                compute_idx(p)
            handles[b] = start_gather(p)
        for b in range(nb - ND, nb):
            with jax.named_scope("gather_wait"):
                handles[b].wait()
            with jax.named_scope("combine"):
                combine_store(b, b % ND)

    return kern


def kernel(inputs, grid, bounds):
    n = inputs.shape[0]
    _, h, w, _ = grid.shape
    scale = (jnp.array([h, w], jnp.float32) - 1.0) / (bounds[1] - bounds[0])
    off = -bounds[0] * scale
    params = jnp.concatenate([scale, off]).reshape(4, 1)
    params = jnp.broadcast_to(params, (4, L)).astype(jnp.float32)
    planes = inputs.T
    table = grid.reshape(-1)
    out = _make_sc_interp(n, h, w)(planes[0], planes[1], table, params)
    return out.reshape(n, 1)
